# baseline (speedup 1.0000x reference)
import functools

import jax
import jax.numpy as jnp
from jax import lax
from jax.experimental import pallas as pl
from jax.experimental.pallas import tpu as pltpu
from jax.experimental.pallas import tpu_sc as plsc

F_CAT = 26
F_NUM = 13
F_TOT = F_CAT + F_NUM
D = 64
EPS = 1e-5


def _make_sc_gather(B: int, V: int):
    info = plsc.get_sparse_core_info()
    nc, ns = info.num_cores, info.num_subcores
    nw = nc * ns
    nrow = F_CAT * D
    assert nrow % nw == 0
    rpw = nrow // nw

    mesh = plsc.VectorSubcoreMesh(core_axis_name="c", subcore_axis_name="s")

    H0 = 49920
    H1 = V - H0

    @functools.partial(
        pl.kernel,
        out_type=jax.ShapeDtypeStruct((nrow, B), jnp.float32),
        mesh=mesh,
        scratch_types=[
            pltpu.VMEM((H0,), jnp.float32),
            pltpu.VMEM((H1,), jnp.float32),
            pltpu.VMEM((B,), jnp.int32),
            pltpu.VMEM((B,), jnp.float32),
            pltpu.VMEM((B,), jnp.float32),
        ]
        + [pltpu.SemaphoreType.DMA] * 4,
        compiler_params=pltpu.CompilerParams(
            use_tc_tiling_on_sc=True, needs_layout_passes=False
        ),
    )
    def sc_gather(
        a2_hbm, idxt_hbm, out_hbm, bufa, bufb, idx_v, vala, valb, *sems
    ):
        sa, sb, soa, sob = sems
        wid = lax.axis_index("s") * nc + lax.axis_index("c")
        r0 = wid * rpw

        def start_h0(r):
            pltpu.async_copy(a2_hbm.at[r, pl.ds(0, H0)], bufa, sa)

        def wait_h0(r):
            pltpu.make_async_copy(
                a2_hbm.at[r, pl.ds(0, H0)], bufa, sa
            ).wait()

        def start_h1(r):
            pltpu.async_copy(
                a2_hbm.at[r, pl.ds(H0, H1)], bufb, sb
            )

        def wait_h1(r):
            pltpu.make_async_copy(
                a2_hbm.at[r, pl.ds(H0, H1)], bufb, sb
            ).wait()

        def load_idx(r):
            f = r // D
            pltpu.sync_copy(idxt_hbm.at[pl.ds(f * B, B)], idx_v)

        def pass0(val_v):
            @pl.loop(0, B // 16, unroll=8)
            def _(j):
                iv = idx_v[pl.ds(j * 16, 16)]
                ivc = jnp.minimum(iv, H0 - 1)
                val_v[pl.ds(j * 16, 16)] = plsc.load_gather(bufa, [ivc])

        def pass1(val_v):
            @pl.loop(0, B // 16, unroll=8)
            def _(j):
                iv = idx_v[pl.ds(j * 16, 16)]
                ivl = iv - H0
                ivc = jnp.maximum(ivl, 0)
                g = plsc.load_gather(bufb, [ivc])
                old = val_v[pl.ds(j * 16, 16)]
                val_v[pl.ds(j * 16, 16)] = jnp.where(ivl >= 0, g, old)

        def start_out(val_v, r, so):
            pltpu.async_copy(val_v, out_hbm.at[r], so)

        def wait_out(val_v, so):
            pltpu.make_async_copy(val_v, out_hbm.at[r0], so).wait()

        load_idx(r0)
        start_h0(r0)

        @pl.loop(0, rpw, step=2)
        def _(k):
            for sub, (val_v, so) in enumerate(((vala, soa), (valb, sob))):
                r = r0 + k + sub
                wait_h0(r)
                start_h1(r)
                pass0(val_v)
                wait_h1(r)
                nr = r + 1

                @pl.when(nr < r0 + rpw)
                def _():
                    start_h0(nr)

                @pl.when(r >= r0 + 2)
                def _():
                    wait_out(val_v, so)

                pass1(val_v)

                @pl.when(nr < r0 + rpw)
                def _():
                    load_idx(nr)

                start_out(val_v, r, so)

        wait_out(vala, soa)
        wait_out(valb, sob)

    return sc_gather


def _epilogue_body(cat_ref, xnt_ref, w_ref, b_ref, g_ref, bt_ref, out_ref):
    cat = cat_ref[...].reshape(F_CAT, D, cat_ref.shape[1])
    xn = xnt_ref[...]
    w = w_ref[...]
    b = b_ref[...]
    num = xn[:, None, :] * w[:, :, None] + b[:, :, None]
    x = jnp.concatenate([cat, num], axis=0)
    mu = jnp.mean(x, axis=1, keepdims=True)
    xc = x - mu
    var = jnp.mean(xc * xc, axis=1, keepdims=True)
    y = xc * lax.rsqrt(var + EPS)
    g = g_ref[...][None, :, :]
    bt = bt_ref[...][None, :, :]
    out_ref[...] = y * g + bt


def _epilogue(cat_t, x_num_t, W_num, b_num, gamma, beta):
    B = cat_t.shape[1]
    BT = 512
    grid = (B // BT,)
    g2 = gamma.reshape(D, 1)
    bt2 = beta.reshape(D, 1)
    return pl.pallas_call(
        _epilogue_body,
        grid=grid,
        in_specs=[
            pl.BlockSpec((F_CAT * D, BT), lambda i: (0, i)),
            pl.BlockSpec((F_NUM, BT), lambda i: (0, i)),
            pl.BlockSpec((F_NUM, D), lambda i: (0, 0)),
            pl.BlockSpec((F_NUM, D), lambda i: (0, 0)),
            pl.BlockSpec((D, 1), lambda i: (0, 0)),
            pl.BlockSpec((D, 1), lambda i: (0, 0)),
        ],
        out_specs=pl.BlockSpec((F_TOT, D, BT), lambda i: (0, 0, i)),
        out_shape=jax.ShapeDtypeStruct((F_TOT, D, B), jnp.float32),
        compiler_params=pltpu.CompilerParams(
            dimension_semantics=("parallel",)
        ),
    )(cat_t, x_num_t, W_num, b_num, g2, bt2)


def kernel(x_cat, x_num, tables, W_num, b_num, gamma, beta):
    B = x_cat.shape[0]
    V = tables.shape[1]
    a2 = jnp.swapaxes(tables, 1, 2).reshape(F_CAT * D, V)
    idxt = x_cat.astype(jnp.int32).T.reshape(F_CAT * B)
    cat_t = _make_sc_gather(B, V)(a2, idxt)
    out_t = _epilogue(cat_t, x_num.T, W_num, b_num, gamma, beta)
    return out_t.transpose(2, 0, 1)


# speedup vs baseline: 1.0066x; 1.0066x over previous
import functools

import jax
import jax.numpy as jnp
from jax import lax
from jax.experimental import pallas as pl
from jax.experimental.pallas import tpu as pltpu
from jax.experimental.pallas import tpu_sc as plsc

F_CAT = 26
F_NUM = 13
F_TOT = F_CAT + F_NUM
D = 64
EPS = 1e-5


def _make_sc_gather(B: int, V: int):
    info = plsc.get_sparse_core_info()
    nc, ns = info.num_cores, info.num_subcores
    nw = nc * ns
    nrow = F_CAT * D
    assert nrow % nw == 0
    rpw = nrow // nw

    mesh = plsc.VectorSubcoreMesh(core_axis_name="c", subcore_axis_name="s")

    @functools.partial(
        pl.kernel,
        out_type=jax.ShapeDtypeStruct((nrow, B), jnp.float32),
        mesh=mesh,
        scratch_types=[
            pltpu.VMEM((V,), jnp.float32),
            pltpu.VMEM((B,), jnp.int32),
            pltpu.VMEM((B,), jnp.float32),
            pltpu.VMEM((B,), jnp.float32),
        ]
        + [pltpu.SemaphoreType.DMA] * 3,
        compiler_params=pltpu.CompilerParams(
            use_tc_tiling_on_sc=True, needs_layout_passes=False
        ),
    )
    def sc_gather(a2_hbm, idxt_hbm, out_hbm, row_v, idx_v, vala, valb, *sems):
        sr, soa, sob = sems
        wid = lax.axis_index("s") * nc + lax.axis_index("c")
        r0 = wid * rpw

        def start_row(r):
            pltpu.async_copy(a2_hbm.at[r], row_v, sr)

        def wait_row(r):
            pltpu.make_async_copy(a2_hbm.at[r], row_v, sr).wait()

        def load_idx(r):
            f = r // D
            pltpu.sync_copy(idxt_hbm.at[pl.ds(f * B, B)], idx_v)

        def start_out(val_v, r, so):
            pltpu.async_copy(val_v, out_hbm.at[r], so)

        def wait_out(val_v, so):
            pltpu.make_async_copy(val_v, out_hbm.at[r0], so).wait()

        start_row(r0)

        @pl.loop(0, rpw, step=2)
        def _(k):
            for val_v, so, sub in ((vala, soa, 0), (valb, sob, 1)):
                r = r0 + k + sub
                load_idx(r)
                wait_row(r)

                @pl.loop(0, B // 16, unroll=8)
                def _(j):
                    iv = idx_v[pl.ds(j * 16, 16)]
                    val_v[pl.ds(j * 16, 16)] = plsc.load_gather(row_v, [iv])

                nr = r + 1

                @pl.when(nr < r0 + rpw)
                def _():
                    start_row(nr)

                @pl.when(r >= r0 + 2)
                def _():
                    wait_out(val_v, so)

                start_out(val_v, r, so)

        wait_out(vala, soa)
        wait_out(valb, sob)

    return sc_gather


def _epilogue_body(cat_ref, xnt_ref, w_ref, b_ref, g_ref, bt_ref, out_ref):
    cat = cat_ref[...].reshape(F_CAT, D, cat_ref.shape[1])
    xn = xnt_ref[...]
    w = w_ref[...]
    b = b_ref[...]
    num = xn[:, None, :] * w[:, :, None] + b[:, :, None]
    x = jnp.concatenate([cat, num], axis=0)
    mu = jnp.mean(x, axis=1, keepdims=True)
    xc = x - mu
    var = jnp.mean(xc * xc, axis=1, keepdims=True)
    y = xc * lax.rsqrt(var + EPS)
    g = g_ref[...][None, :, :]
    bt = bt_ref[...][None, :, :]
    out_ref[...] = y * g + bt


def _epilogue(cat_t, x_num_t, W_num, b_num, gamma, beta):
    B = cat_t.shape[1]
    BT = 512
    grid = (B // BT,)
    g2 = gamma.reshape(D, 1)
    bt2 = beta.reshape(D, 1)
    return pl.pallas_call(
        _epilogue_body,
        grid=grid,
        in_specs=[
            pl.BlockSpec((F_CAT * D, BT), lambda i: (0, i)),
            pl.BlockSpec((F_NUM, BT), lambda i: (0, i)),
            pl.BlockSpec((F_NUM, D), lambda i: (0, 0)),
            pl.BlockSpec((F_NUM, D), lambda i: (0, 0)),
            pl.BlockSpec((D, 1), lambda i: (0, 0)),
            pl.BlockSpec((D, 1), lambda i: (0, 0)),
        ],
        out_specs=pl.BlockSpec((F_TOT, D, BT), lambda i: (0, 0, i)),
        out_shape=jax.ShapeDtypeStruct((F_TOT, D, B), jnp.float32),
        compiler_params=pltpu.CompilerParams(
            dimension_semantics=("parallel",)
        ),
    )(cat_t, x_num_t, W_num, b_num, g2, bt2)


def kernel(x_cat, x_num, tables, W_num, b_num, gamma, beta):
    B = x_cat.shape[0]
    V = tables.shape[1]
    a2 = jnp.swapaxes(tables, 1, 2).reshape(F_CAT * D, V)
    idxt = x_cat.astype(jnp.int32).T.reshape(F_CAT * B)
    cat_t = _make_sc_gather(B, V)(a2, idxt)
    out_t = _epilogue(cat_t, x_num.T, W_num, b_num, gamma, beta)
    return out_t.transpose(2, 0, 1)


# speedup vs baseline: 1.1624x; 1.1548x over previous
import functools

import jax
import jax.numpy as jnp
from jax import lax
from jax.experimental import pallas as pl
from jax.experimental.pallas import tpu as pltpu
from jax.experimental.pallas import tpu_sc as plsc

F_CAT = 26
F_NUM = 13
F_TOT = F_CAT + F_NUM
D = 64
EPS = 1e-5


def _make_sc_gather(B: int, V: int):
    info = plsc.get_sparse_core_info()
    nc, ns = info.num_cores, info.num_subcores
    nw = nc * ns
    nrow = F_CAT * D
    assert nrow % nw == 0
    rpw = nrow // nw

    mesh = plsc.VectorSubcoreMesh(core_axis_name="c", subcore_axis_name="s")

    @functools.partial(
        pl.kernel,
        out_type=jax.ShapeDtypeStruct((nrow, B), jnp.float32),
        mesh=mesh,
        scratch_types=[
            pltpu.VMEM((V,), jnp.float32),
            pltpu.VMEM((B,), jnp.int32),
            pltpu.VMEM((B,), jnp.float32),
            pltpu.VMEM((B,), jnp.float32),
        ]
        + [pltpu.SemaphoreType.DMA] * 3,
        compiler_params=pltpu.CompilerParams(
            use_tc_tiling_on_sc=True, needs_layout_passes=False
        ),
    )
    def sc_gather(a2_hbm, idxt_hbm, out_hbm, row_v, idx_v, vala, valb, *sems):
        sr, soa, sob = sems
        wid = lax.axis_index("s") * nc + lax.axis_index("c")
        r0 = wid * rpw

        def start_row(r):
            pltpu.async_copy(a2_hbm.at[r], row_v, sr)

        def wait_row(r):
            pltpu.make_async_copy(a2_hbm.at[r], row_v, sr).wait()

        def load_idx(r):
            f = r // D
            pltpu.sync_copy(idxt_hbm.at[pl.ds(f * B, B)], idx_v)

        def start_out(val_v, r, so):
            pltpu.async_copy(val_v, out_hbm.at[r], so)

        def wait_out(val_v, so):
            pltpu.make_async_copy(val_v, out_hbm.at[r0], so).wait()

        start_row(r0)

        @pl.loop(0, rpw, step=2)
        def _(k):
            for val_v, so, sub in ((vala, soa, 0), (valb, sob, 1)):
                r = r0 + k + sub
                load_idx(r)
                wait_row(r)

                for j in range(B // 16):
                    iv = idx_v[pl.ds(j * 16, 16)]
                    val_v[pl.ds(j * 16, 16)] = plsc.load_gather(row_v, [iv])

                nr = r + 1

                @pl.when(nr < r0 + rpw)
                def _():
                    start_row(nr)

                @pl.when(r >= r0 + 2)
                def _():
                    wait_out(val_v, so)

                start_out(val_v, r, so)

        wait_out(vala, soa)
        wait_out(valb, sob)

    return sc_gather


def _epilogue_body(cat_ref, xnt_ref, w_ref, b_ref, g_ref, bt_ref, out_ref):
    cat = cat_ref[...].reshape(F_CAT, D, cat_ref.shape[1])
    xn = xnt_ref[...]
    w = w_ref[...]
    b = b_ref[...]
    num = xn[:, None, :] * w[:, :, None] + b[:, :, None]
    x = jnp.concatenate([cat, num], axis=0)
    mu = jnp.mean(x, axis=1, keepdims=True)
    xc = x - mu
    var = jnp.mean(xc * xc, axis=1, keepdims=True)
    y = xc * lax.rsqrt(var + EPS)
    g = g_ref[...][None, :, :]
    bt = bt_ref[...][None, :, :]
    out_ref[...] = y * g + bt


def _epilogue(cat_t, x_num_t, W_num, b_num, gamma, beta):
    B = cat_t.shape[1]
    BT = 512
    grid = (B // BT,)
    g2 = gamma.reshape(D, 1)
    bt2 = beta.reshape(D, 1)
    return pl.pallas_call(
        _epilogue_body,
        grid=grid,
        in_specs=[
            pl.BlockSpec((F_CAT * D, BT), lambda i: (0, i)),
            pl.BlockSpec((F_NUM, BT), lambda i: (0, i)),
            pl.BlockSpec((F_NUM, D), lambda i: (0, 0)),
            pl.BlockSpec((F_NUM, D), lambda i: (0, 0)),
            pl.BlockSpec((D, 1), lambda i: (0, 0)),
            pl.BlockSpec((D, 1), lambda i: (0, 0)),
        ],
        out_specs=pl.BlockSpec((F_TOT, D, BT), lambda i: (0, 0, i)),
        out_shape=jax.ShapeDtypeStruct((F_TOT, D, B), jnp.float32),
        compiler_params=pltpu.CompilerParams(
            dimension_semantics=("parallel",)
        ),
    )(cat_t, x_num_t, W_num, b_num, g2, bt2)


def kernel(x_cat, x_num, tables, W_num, b_num, gamma, beta):
    B = x_cat.shape[0]
    V = tables.shape[1]
    a2 = jnp.swapaxes(tables, 1, 2).reshape(F_CAT * D, V)
    idxt = x_cat.astype(jnp.int32).T.reshape(F_CAT * B)
    cat_t = _make_sc_gather(B, V)(a2, idxt)
    out_t = _epilogue(cat_t, x_num.T, W_num, b_num, gamma, beta)
    return out_t.transpose(2, 0, 1)


# speedup vs baseline: 1.1746x; 1.0105x over previous
import functools

import jax
import jax.numpy as jnp
from jax import lax
from jax.experimental import pallas as pl
from jax.experimental.pallas import tpu as pltpu
from jax.experimental.pallas import tpu_sc as plsc

F_CAT = 26
F_NUM = 13
F_TOT = F_CAT + F_NUM
D = 64
EPS = 1e-5


def _make_sc_gather(B: int, V: int):
    info = plsc.get_sparse_core_info()
    nc, ns = info.num_cores, info.num_subcores
    nw = nc * ns
    nrow = F_CAT * D
    assert nrow % nw == 0
    rpw = nrow // nw

    mesh = plsc.VectorSubcoreMesh(core_axis_name="c", subcore_axis_name="s")

    @functools.partial(
        pl.kernel,
        out_type=jax.ShapeDtypeStruct((nrow, B), jnp.float32),
        mesh=mesh,
        scratch_types=[
            pltpu.VMEM((V,), jnp.float32),
            pltpu.VMEM((B,), jnp.int32),
            pltpu.VMEM((B,), jnp.float32),
            pltpu.VMEM((B,), jnp.float32),
        ]
        + [pltpu.SemaphoreType.DMA] * 3,
        compiler_params=pltpu.CompilerParams(
            use_tc_tiling_on_sc=True, needs_layout_passes=False
        ),
    )
    def sc_gather(a2_hbm, idxt_hbm, out_hbm, row_v, idx_v, vala, valb, *sems):
        sr, soa, sob = sems
        wid = lax.axis_index("s") * nc + lax.axis_index("c")
        r0 = wid * rpw

        def start_row(r):
            pltpu.async_copy(a2_hbm.at[r], row_v, sr)

        def wait_row(r):
            pltpu.make_async_copy(a2_hbm.at[r], row_v, sr).wait()

        def load_idx(r):
            f = r // D
            pltpu.sync_copy(idxt_hbm.at[pl.ds(f * B, B)], idx_v)

        def start_out(val_v, r, so):
            pltpu.async_copy(val_v, out_hbm.at[r], so)

        def wait_out(val_v, so):
            pltpu.make_async_copy(val_v, out_hbm.at[r0], so).wait()

        start_row(r0)

        @pl.loop(0, rpw, step=2)
        def _(k):
            for val_v, so, sub in ((vala, soa, 0), (valb, sob, 1)):
                r = r0 + k + sub

                @pl.when(jnp.logical_or(r == r0, lax.rem(r, D) == 0))
                def _():
                    load_idx(r)
                wait_row(r)

                for j in range(B // 16):
                    iv = idx_v[pl.ds(j * 16, 16)]
                    val_v[pl.ds(j * 16, 16)] = plsc.load_gather(row_v, [iv])

                nr = r + 1

                @pl.when(nr < r0 + rpw)
                def _():
                    start_row(nr)

                @pl.when(r >= r0 + 2)
                def _():
                    wait_out(val_v, so)

                start_out(val_v, r, so)

        wait_out(vala, soa)
        wait_out(valb, sob)

    return sc_gather


def _epilogue_body(cat_ref, xnt_ref, w_ref, b_ref, g_ref, bt_ref, out_ref):
    cat = cat_ref[...].reshape(F_CAT, D, cat_ref.shape[1])
    xn = xnt_ref[...]
    w = w_ref[...]
    b = b_ref[...]
    num = xn[:, None, :] * w[:, :, None] + b[:, :, None]
    x = jnp.concatenate([cat, num], axis=0)
    mu = jnp.mean(x, axis=1, keepdims=True)
    xc = x - mu
    var = jnp.mean(xc * xc, axis=1, keepdims=True)
    y = xc * lax.rsqrt(var + EPS)
    g = g_ref[...][None, :, :]
    bt = bt_ref[...][None, :, :]
    out_ref[...] = y * g + bt


def _epilogue(cat_t, x_num_t, W_num, b_num, gamma, beta):
    B = cat_t.shape[1]
    BT = 1024
    grid = (B // BT,)
    g2 = gamma.reshape(D, 1)
    bt2 = beta.reshape(D, 1)
    return pl.pallas_call(
        _epilogue_body,
        grid=grid,
        in_specs=[
            pl.BlockSpec((F_CAT * D, BT), lambda i: (0, i)),
            pl.BlockSpec((F_NUM, BT), lambda i: (0, i)),
            pl.BlockSpec((F_NUM, D), lambda i: (0, 0)),
            pl.BlockSpec((F_NUM, D), lambda i: (0, 0)),
            pl.BlockSpec((D, 1), lambda i: (0, 0)),
            pl.BlockSpec((D, 1), lambda i: (0, 0)),
        ],
        out_specs=pl.BlockSpec((F_TOT, D, BT), lambda i: (0, 0, i)),
        out_shape=jax.ShapeDtypeStruct((F_TOT, D, B), jnp.float32),
        compiler_params=pltpu.CompilerParams(
            dimension_semantics=("parallel",)
        ),
    )(cat_t, x_num_t, W_num, b_num, g2, bt2)


def kernel(x_cat, x_num, tables, W_num, b_num, gamma, beta):
    B = x_cat.shape[0]
    V = tables.shape[1]
    a2 = jnp.swapaxes(tables, 1, 2).reshape(F_CAT * D, V)
    idxt = x_cat.astype(jnp.int32).T.reshape(F_CAT * B)
    cat_t = _make_sc_gather(B, V)(a2, idxt)
    out_t = _epilogue(cat_t, x_num.T, W_num, b_num, gamma, beta)
    return out_t.transpose(2, 0, 1)
